# Initial kernel scaffold; baseline (speedup 1.0000x reference)
import jax
import jax.numpy as jnp
from jax.experimental import pallas as pl


def kernel(reg, priors, obj, levels):
    raise NotImplementedError("write your pallas kernel here")


# speedup vs baseline: 7.0402x; 7.0402x over previous
import math

import jax
import jax.numpy as jnp
from jax.experimental import pallas as pl
from jax.experimental.pallas import tpu as pltpu

_K = 1000
_P = 1024
_IOU_THR = 0.7
_CANVAS_H = 800.0
_CANVAS_W = 1333.0
_BBOX_CLIP = math.log(1000.0 / 16.0)
_LVL_OFF = _CANVAS_W + _CANVAS_H


def _nms_kernel(in_ref, out_ref, m_ref):
    data = in_ref[:]
    dx, dy = data[0:1], data[1:2]
    dw, dh = data[2:3], data[3:4]
    px1, py1 = data[4:5], data[5:6]
    px2, py2 = data[6:7], data[7:8]
    sc, lv = data[8:9], data[9:10]

    ws = px2 - px1
    hs = py2 - py1
    cx = px1 + 0.5 * ws
    cy = py1 + 0.5 * hs
    dw = jnp.minimum(dw, _BBOX_CLIP)
    dh = jnp.minimum(dh, _BBOX_CLIP)
    pcx = dx * ws + cx
    pcy = dy * hs + cy
    pw = jnp.exp(dw) * ws
    ph = jnp.exp(dh) * hs
    x1 = jnp.clip(pcx - 0.5 * pw, 0.0, _CANVAS_W)
    y1 = jnp.clip(pcy - 0.5 * ph, 0.0, _CANVAS_H)
    x2 = jnp.clip(pcx + 0.5 * pw, 0.0, _CANVAS_W)
    y2 = jnp.clip(pcy + 0.5 * ph, 0.0, _CANVAS_H)

    small = ((x2 - x1) < 1e-2) | ((y2 - y1) < 1e-2)
    scores = jnp.where(small, -1.0, sc)

    off = lv * _LVL_OFF
    ox1, oy1 = x1 + off, y1 + off
    ox2, oy2 = x2 + off, y2 + off
    area = (ox2 - ox1) * (oy2 - oy1)

    i0 = jax.lax.broadcasted_iota(jnp.int32, (_P, _P), 0)
    i1 = jax.lax.broadcasted_iota(jnp.int32, (_P, _P), 1)
    ident = (i0 == i1).astype(jnp.float32)

    cat5 = jnp.concatenate([ox1, oy1, ox2, oy2, area], axis=0)
    cols = jax.lax.dot_general(
        ident, cat5, (((1,), (1,)), ((), ())),
        preferred_element_type=jnp.float32,
        precision=jax.lax.Precision.HIGHEST)

    R = 256
    for c in range(_P // R):
        a_x1 = cols[c * R:(c + 1) * R, 0:1]
        a_y1 = cols[c * R:(c + 1) * R, 1:2]
        a_x2 = cols[c * R:(c + 1) * R, 2:3]
        a_y2 = cols[c * R:(c + 1) * R, 3:4]
        a_area = cols[c * R:(c + 1) * R, 4:5]
        ltx = jnp.maximum(a_x1, ox1)
        lty = jnp.maximum(a_y1, oy1)
        rbx = jnp.minimum(a_x2, ox2)
        rby = jnp.minimum(a_y2, oy2)
        wv = jnp.maximum(rbx - ltx, 0.0)
        hv = jnp.maximum(rby - lty, 0.0)
        inter = wv * hv
        union = a_area + area - inter
        iou = inter / jnp.maximum(union, 1e-9)
        ir = jax.lax.broadcasted_iota(jnp.int32, (R, _P), 0) + c * R
        jr = jax.lax.broadcasted_iota(jnp.int32, (R, _P), 1)
        mblk = ((iou > _IOU_THR) & (jr > ir)).astype(jnp.float32)
        m_ref[c * R:(c + 1) * R] = mblk.reshape(R, 1, _P)

    lane = jax.lax.broadcasted_iota(jnp.int32, (1, _P), 1)
    sup0 = jnp.where(small, 1.0, 0.0)

    def body(i, sup):
        row = m_ref[pl.ds(i, 1)].reshape(1, _P)
        e = (lane == i).astype(jnp.float32)
        sup_i = jnp.sum(sup * e)
        return jnp.maximum(sup, row * (1.0 - sup_i))

    sup = jax.lax.fori_loop(0, _K, body, sup0)
    keep = 1.0 - sup

    rows = jnp.concatenate(
        [x1 * keep, y1 * keep, x2 * keep, y2 * keep, scores * keep,
         jnp.zeros((3, _P), jnp.float32)], axis=0)

    small_f = jnp.where(small, 1.0, 0.0)
    valid_f = 1.0 - small_f
    ustrict = (i0 < i1).astype(jnp.float32)
    ex_valid = jax.lax.dot_general(
        valid_f, ustrict, (((1,), (0,)), ((), ())),
        preferred_element_type=jnp.float32,
        precision=jax.lax.Precision.HIGHEST)
    ex_small = jax.lax.dot_general(
        small_f, ustrict, (((1,), (0,)), ((), ())),
        preferred_element_type=jnp.float32,
        precision=jax.lax.Precision.HIGHEST)
    nvalid = jnp.sum(valid_f)
    dest = jnp.where(small, nvalid + ex_small, ex_valid)

    jrow = jax.lax.broadcasted_iota(jnp.int32, (_P, _P), 0).astype(jnp.float32)
    perm_t = (jrow == dest).astype(jnp.float32)
    out_ref[:] = jax.lax.dot_general(
        rows, perm_t, (((1,), (1,)), ((), ())),
        preferred_element_type=jnp.float32,
        precision=jax.lax.Precision.HIGHEST)


def kernel(reg, priors, obj, levels):
    scores0, idx = jax.lax.top_k(obj, _K)
    reg_k = jnp.take(reg, idx, axis=0)
    pri_k = jnp.take(priors, idx, axis=0)
    lv_k = jnp.take(levels, idx, axis=0).astype(jnp.float32)

    pad = _P - _K
    reg_t = jnp.pad(reg_k, ((0, pad), (0, 0))).T
    pri_t = jnp.pad(pri_k, ((0, pad), (0, 0))).T
    sc_p = jnp.pad(scores0, (0, pad))[None]
    lv_p = jnp.pad(lv_k, (0, pad))[None]
    packed = jnp.concatenate(
        [reg_t, pri_t, sc_p, lv_p, jnp.zeros((6, _P), jnp.float32)], axis=0)

    out_t = pl.pallas_call(
        _nms_kernel,
        out_shape=jax.ShapeDtypeStruct((8, _P), jnp.float32),
        scratch_shapes=[pltpu.VMEM((_P, 1, _P), jnp.float32)],
    )(packed)
    return out_t[:5, :_K].T


# speedup vs baseline: 21.9481x; 3.1175x over previous
import math

import jax
import jax.numpy as jnp
from jax.experimental import pallas as pl
from jax.experimental.pallas import tpu as pltpu

_K = 1000
_P = 1024
_IOU_THR = 0.7
_CANVAS_H = 800.0
_CANVAS_W = 1333.0
_BBOX_CLIP = math.log(1000.0 / 16.0)
_LVL_OFF = _CANVAS_W + _CANVAS_H


def _nms_kernel(in_ref, out_ref, m_ref):
    data = in_ref[:]
    dx, dy = data[0:1], data[1:2]
    dw, dh = data[2:3], data[3:4]
    px1, py1 = data[4:5], data[5:6]
    px2, py2 = data[6:7], data[7:8]
    sc, lv = data[8:9], data[9:10]

    ws = px2 - px1
    hs = py2 - py1
    cx = px1 + 0.5 * ws
    cy = py1 + 0.5 * hs
    dw = jnp.minimum(dw, _BBOX_CLIP)
    dh = jnp.minimum(dh, _BBOX_CLIP)
    pcx = dx * ws + cx
    pcy = dy * hs + cy
    pw = jnp.exp(dw) * ws
    ph = jnp.exp(dh) * hs
    x1 = jnp.clip(pcx - 0.5 * pw, 0.0, _CANVAS_W)
    y1 = jnp.clip(pcy - 0.5 * ph, 0.0, _CANVAS_H)
    x2 = jnp.clip(pcx + 0.5 * pw, 0.0, _CANVAS_W)
    y2 = jnp.clip(pcy + 0.5 * ph, 0.0, _CANVAS_H)

    small = ((x2 - x1) < 1e-2) | ((y2 - y1) < 1e-2)
    scores = jnp.where(small, -1.0, sc)

    off = lv * _LVL_OFF
    ox1, oy1 = x1 + off, y1 + off
    ox2, oy2 = x2 + off, y2 + off
    area = (ox2 - ox1) * (oy2 - oy1)

    i0 = jax.lax.broadcasted_iota(jnp.int32, (_P, _P), 0)
    i1 = jax.lax.broadcasted_iota(jnp.int32, (_P, _P), 1)
    ident = (i0 == i1).astype(jnp.float32)

    cat5 = jnp.concatenate([ox1, oy1, ox2, oy2, area], axis=0)
    cols = jax.lax.dot_general(
        ident, cat5, (((1,), (1,)), ((), ())),
        preferred_element_type=jnp.float32,
        precision=jax.lax.Precision.HIGHEST)

    R = 256
    for c in range(_P // R):
        a_x1 = cols[c * R:(c + 1) * R, 0:1]
        a_y1 = cols[c * R:(c + 1) * R, 1:2]
        a_x2 = cols[c * R:(c + 1) * R, 2:3]
        a_y2 = cols[c * R:(c + 1) * R, 3:4]
        a_area = cols[c * R:(c + 1) * R, 4:5]
        ltx = jnp.maximum(a_x1, ox1)
        lty = jnp.maximum(a_y1, oy1)
        rbx = jnp.minimum(a_x2, ox2)
        rby = jnp.minimum(a_y2, oy2)
        wv = jnp.maximum(rbx - ltx, 0.0)
        hv = jnp.maximum(rby - lty, 0.0)
        inter = wv * hv
        union = a_area + area - inter
        iou = inter / jnp.maximum(union, 1e-9)
        ir = jax.lax.broadcasted_iota(jnp.int32, (R, _P), 0) + c * R
        jr = jax.lax.broadcasted_iota(jnp.int32, (R, _P), 1)
        mblk = ((iou > _IOU_THR) & (jr > ir)).astype(jnp.float32)
        m_ref[c * R:(c + 1) * R] = mblk

    sup0 = jnp.where(small, 1.0, 0.0)
    init_keep = 1.0 - sup0

    def fp_cond(carry):
        return carry[1]

    def fp_body(carry):
        k, _ = carry
        hit = jax.lax.dot_general(
            k, m_ref[:], (((1,), (0,)), ((), ())),
            preferred_element_type=jnp.float32)
        newk = init_keep * jnp.where(hit > 0.0, 0.0, 1.0)
        return newk, jnp.any(newk != k)

    keep, _ = jax.lax.while_loop(
        fp_cond, fp_body, (init_keep, jnp.bool_(True)))

    rows = jnp.concatenate(
        [x1 * keep, y1 * keep, x2 * keep, y2 * keep, scores * keep,
         jnp.zeros((3, _P), jnp.float32)], axis=0)

    small_f = jnp.where(small, 1.0, 0.0)
    valid_f = 1.0 - small_f
    ustrict = (i0 < i1).astype(jnp.float32)
    ex_valid = jax.lax.dot_general(
        valid_f, ustrict, (((1,), (0,)), ((), ())),
        preferred_element_type=jnp.float32,
        precision=jax.lax.Precision.HIGHEST)
    ex_small = jax.lax.dot_general(
        small_f, ustrict, (((1,), (0,)), ((), ())),
        preferred_element_type=jnp.float32,
        precision=jax.lax.Precision.HIGHEST)
    nvalid = jnp.sum(valid_f)
    dest = jnp.where(small, nvalid + ex_small, ex_valid)

    jrow = jax.lax.broadcasted_iota(jnp.int32, (_P, _P), 0).astype(jnp.float32)
    perm_t = (jrow == dest).astype(jnp.float32)
    out_ref[:] = jax.lax.dot_general(
        rows, perm_t, (((1,), (1,)), ((), ())),
        preferred_element_type=jnp.float32,
        precision=jax.lax.Precision.HIGHEST)


def kernel(reg, priors, obj, levels):
    scores0, idx = jax.lax.top_k(obj, _K)
    reg_k = jnp.take(reg, idx, axis=0)
    pri_k = jnp.take(priors, idx, axis=0)
    lv_k = jnp.take(levels, idx, axis=0).astype(jnp.float32)

    pad = _P - _K
    reg_t = jnp.pad(reg_k, ((0, pad), (0, 0))).T
    pri_t = jnp.pad(pri_k, ((0, pad), (0, 0))).T
    sc_p = jnp.pad(scores0, (0, pad))[None]
    lv_p = jnp.pad(lv_k, (0, pad))[None]
    packed = jnp.concatenate(
        [reg_t, pri_t, sc_p, lv_p, jnp.zeros((6, _P), jnp.float32)], axis=0)

    out_t = pl.pallas_call(
        _nms_kernel,
        out_shape=jax.ShapeDtypeStruct((8, _P), jnp.float32),
        scratch_shapes=[pltpu.VMEM((_P, _P), jnp.float32)],
    )(packed)
    return out_t[:5, :_K].T
